# speedup vs baseline: 1.5927x; 1.5927x over previous
import jax
import jax.numpy as jnp
from jax.experimental import pallas as pl
from jax.experimental.pallas import tpu as pltpu


def _mlp_fold_kernel(x_ref, w1_ref, b1_ref, w2_ref, b2_ref, w3_ref, b3_ref,
                     s_ref, means_ref, stds_ref):
    x = x_ref[...]
    h1 = jnp.dot(x, w1_ref[...], preferred_element_type=jnp.float32)
    h1 = jnp.maximum(h1 + b1_ref[...], 0.0)
    h2 = jnp.dot(h1, w2_ref[...], preferred_element_type=jnp.float32)
    h2 = jnp.maximum(h2 + b2_ref[...], 0.0)
    m = jnp.dot(h2, w3_ref[...], preferred_element_type=jnp.float32)
    means_ref[...] = (m + b3_ref[...]).astype(means_ref.dtype)
    stds_ref[...] = jnp.broadcast_to(s_ref[...], stds_ref.shape)


def kernel(x, w1, b1, w2, b2, w3, b3, logstds):
    batch, state_size = x.shape
    action_size = w3.shape[1]

    fold = 16
    while batch % fold:
        fold //= 2
    rows = batch // fold

    eye = jnp.eye(fold, dtype=w1.dtype)
    w1b = jnp.kron(eye, w1)
    w2b = jnp.kron(eye, w2)
    w3b = jnp.kron(eye, w3)
    b1b = jnp.tile(b1, (1, fold))
    b2b = jnp.tile(b2, (1, fold))
    b3b = jnp.tile(b3, (1, fold))
    srow = jnp.tile(jnp.minimum(jnp.exp(logstds), 10.0), (1, fold))

    xf = x.reshape(rows, fold * state_size)

    row_block = 4096
    while rows % row_block:
        row_block //= 2
    grid = rows // row_block

    const = lambda shape: pl.BlockSpec(shape, lambda i: (0, 0))
    out_shape = jax.ShapeDtypeStruct((rows, fold * action_size), jnp.float32)
    means, stds = pl.pallas_call(
        _mlp_fold_kernel,
        out_shape=(out_shape, out_shape),
        grid=(grid,),
        in_specs=[
            pl.BlockSpec((row_block, fold * state_size), lambda i: (i, 0)),
            const(w1b.shape), const(b1b.shape),
            const(w2b.shape), const(b2b.shape),
            const(w3b.shape), const(b3b.shape),
            const(srow.shape),
        ],
        out_specs=(
            pl.BlockSpec((row_block, fold * action_size), lambda i: (i, 0)),
            pl.BlockSpec((row_block, fold * action_size), lambda i: (i, 0)),
        ),
        compiler_params=pltpu.CompilerParams(
            dimension_semantics=("parallel",)),
    )(xf, w1b, b1b, w2b, b2b, w3b, b3b, srow)

    return (means.reshape(batch, action_size),
            stds.reshape(batch, action_size))


# speedup vs baseline: 46.7243x; 29.3361x over previous
import jax
import jax.numpy as jnp
from jax.experimental import pallas as pl
from jax.experimental.pallas import tpu as pltpu


def _actor_t_kernel(x_ref, w1_ref, b1_ref, w2_ref, b2_ref, w3_ref, b3_ref,
                    s_ref, means_ref, stds_ref):
    x = x_ref[...]
    h1 = jnp.dot(w1_ref[...], x, preferred_element_type=jnp.float32)
    h1 = jnp.maximum(h1 + b1_ref[...], 0.0)
    h2 = jnp.dot(w2_ref[...], h1, preferred_element_type=jnp.float32)
    h2 = jnp.maximum(h2 + b2_ref[...], 0.0)
    m = jnp.dot(w3_ref[...], h2, preferred_element_type=jnp.float32)
    means_ref[...] = (m + b3_ref[...]).astype(means_ref.dtype)
    stds_ref[...] = jnp.broadcast_to(s_ref[...], stds_ref.shape)


def kernel(x, w1, b1, w2, b2, w3, b3, logstds):
    batch, state_size = x.shape
    hidden1 = w1.shape[1]
    hidden2 = w2.shape[1]
    action_size = w3.shape[1]

    xt = x.T
    w1t, w2t, w3t = w1.T, w2.T, w3.T
    b1t, b2t, b3t = b1.T, b2.T, b3.T
    st = jnp.minimum(jnp.exp(logstds), 10.0).T

    n_block = 65536
    while batch % n_block:
        n_block //= 2
    grid = batch // n_block

    const = lambda shape: pl.BlockSpec(shape, lambda i: (0, 0))
    out_t = jax.ShapeDtypeStruct((action_size, batch), jnp.float32)
    means_t, stds_t = pl.pallas_call(
        _actor_t_kernel,
        out_shape=(out_t, out_t),
        grid=(grid,),
        in_specs=[
            pl.BlockSpec((state_size, n_block), lambda i: (0, i)),
            const(w1t.shape), const(b1t.shape),
            const(w2t.shape), const(b2t.shape),
            const(w3t.shape), const(b3t.shape),
            const(st.shape),
        ],
        out_specs=(
            pl.BlockSpec((action_size, n_block), lambda i: (0, i)),
            pl.BlockSpec((action_size, n_block), lambda i: (0, i)),
        ),
        compiler_params=pltpu.CompilerParams(
            dimension_semantics=("parallel",)),
    )(xt, w1t, b1t, w2t, b2t, w3t, b3t, st)

    return means_t.T, stds_t.T


# speedup vs baseline: 47.9644x; 1.0265x over previous
import jax
import jax.numpy as jnp
from jax.experimental import pallas as pl
from jax.experimental.pallas import tpu as pltpu


def _actor_t_kernel(x_ref, w1_ref, b1_ref, w2_ref, b2_ref, w3_ref, b3_ref,
                    s_ref, means_ref, stds_ref):
    x = x_ref[...]
    h1 = jnp.dot(w1_ref[...], x, preferred_element_type=jnp.float32)
    h1 = jnp.maximum(h1 + b1_ref[...], 0.0)
    h2 = jnp.dot(w2_ref[...], h1, preferred_element_type=jnp.float32)
    h2 = jnp.maximum(h2 + b2_ref[...], 0.0)
    m = jnp.dot(w3_ref[...], h2, preferred_element_type=jnp.float32)
    means_ref[...] = (m + b3_ref[...]).astype(means_ref.dtype)
    stds_ref[...] = jnp.broadcast_to(s_ref[...], stds_ref.shape)


def kernel(x, w1, b1, w2, b2, w3, b3, logstds):
    batch, state_size = x.shape
    hidden1 = w1.shape[1]
    hidden2 = w2.shape[1]
    action_size = w3.shape[1]

    xt = x.T
    w1t, w2t, w3t = w1.T, w2.T, w3.T
    b1t, b2t, b3t = b1.T, b2.T, b3.T
    st = jnp.minimum(jnp.exp(logstds), 10.0).T

    n_block = 131072
    while batch % n_block:
        n_block //= 2
    grid = batch // n_block

    const = lambda shape: pl.BlockSpec(shape, lambda i: (0, 0))
    out_t = jax.ShapeDtypeStruct((action_size, batch), jnp.float32)
    means_t, stds_t = pl.pallas_call(
        _actor_t_kernel,
        out_shape=(out_t, out_t),
        grid=(grid,),
        in_specs=[
            pl.BlockSpec((state_size, n_block), lambda i: (0, i)),
            const(w1t.shape), const(b1t.shape),
            const(w2t.shape), const(b2t.shape),
            const(w3t.shape), const(b3t.shape),
            const(st.shape),
        ],
        out_specs=(
            pl.BlockSpec((action_size, n_block), lambda i: (0, i)),
            pl.BlockSpec((action_size, n_block), lambda i: (0, i)),
        ),
        compiler_params=pltpu.CompilerParams(
            dimension_semantics=("parallel",)),
    )(xt, w1t, b1t, w2t, b2t, w3t, b3t, st)

    return means_t.T, stds_t.T
